# baseline (speedup 1.0000x reference)
import functools

import jax
import jax.numpy as jnp
from jax import lax
from jax.experimental import pallas as pl
from jax.experimental.pallas import tpu as pltpu
from jax.experimental.pallas import tpu_sc as plsc

VOCAB = 100000
EMBED_DIM = 128
BATCH = 4096
FIELDS = 26

TOT = BATCH * FIELDS
NUM_CORES = 2
NUM_SUBCORES = 16
NW = NUM_CORES * NUM_SUBCORES
B_PER_W = TOT // NW
CHUNK = 416
N_CHUNKS = B_PER_W // CHUNK
NBUF = 2

_mesh = plsc.VectorSubcoreMesh(core_axis_name="c", subcore_axis_name="s")

_scratch = (
    [pltpu.VMEM((B_PER_W,), jnp.int32)]
    + [pltpu.VMEM((CHUNK, EMBED_DIM), jnp.float32) for _ in range(NBUF)]
    + [pltpu.SemaphoreType.DMA for _ in range(2 * NBUF)]
)


@functools.partial(
    pl.kernel,
    mesh=_mesh,
    out_type=jax.ShapeDtypeStruct((TOT, EMBED_DIM), jnp.float32),
    scratch_types=_scratch,
)
def _embedding_gather(idx_hbm, table_hbm, out_hbm, idx_v, *bufs):
    rows = bufs[:NBUF]
    gsem = bufs[NBUF : 2 * NBUF]
    osem = bufs[2 * NBUF : 3 * NBUF]
    wid = lax.axis_index("s") * NUM_CORES + lax.axis_index("c")
    base = wid * B_PER_W

    pltpu.sync_copy(idx_hbm.at[pl.ds(base, B_PER_W)], idx_v)

    def gather_start(c):
        b = c % NBUF
        return pltpu.async_copy(
            table_hbm.at[idx_v.at[pl.ds(c * CHUNK, CHUNK)]], rows[b], gsem[b]
        )

    def writeback_start(c):
        b = c % NBUF
        return pltpu.async_copy(
            rows[b], out_hbm.at[pl.ds(base + c * CHUNK, CHUNK)], osem[b]
        )

    g = [None] * N_CHUNKS
    wb = [None] * NBUF
    for c in range(min(NBUF - 1, N_CHUNKS)):
        g[c] = gather_start(c)
    for c in range(N_CHUNKS):
        nxt = c + NBUF - 1
        if nxt < N_CHUNKS:
            b = nxt % NBUF
            if wb[b] is not None:
                wb[b].wait()
            g[nxt] = gather_start(nxt)
        g[c].wait()
        wb[c % NBUF] = writeback_start(c)
    for h in wb:
        if h is not None:
            h.wait()


def kernel(input, weight):
    idx = input.T.reshape(-1).astype(jnp.int32)
    out = _embedding_gather(idx, weight)
    return out.reshape(FIELDS, BATCH, EMBED_DIM).transpose(1, 0, 2)
